# baseline (device time: 471244 ns/iter reference)
import jax
import jax.numpy as jnp
from jax import lax
from jax.experimental import pallas as pl
from jax.experimental.pallas import tpu as pltpu

N_DEV = 16
SQ = 2048
SKV = 2048
D_MODEL = 1024
H_PER = 8
DH = 128
SCALE = 0.08838834764831843
CHUNK = SQ // N_DEV


def _ring_allreduce(partial):
    rows, cols = partial.shape

    def body(p_ref, out_ref, rs_scratch, send_sem, rs_recv_sems, ag_recv_sems):
        me = lax.axis_index("i")
        right = lax.rem(me + 1, N_DEV)
        left = lax.rem(me + N_DEV - 1, N_DEV)

        barrier_sem = pltpu.get_barrier_semaphore()
        for nbr in (left, right):
            pl.semaphore_signal(
                barrier_sem, inc=1,
                device_id=(nbr,), device_id_type=pl.DeviceIdType.MESH,
            )
        pl.semaphore_wait(barrier_sem, 2)

        out_ref[...] = p_ref[...]

        for s in range(N_DEV - 1):
            send_c = lax.rem(me - s + 2 * N_DEV, N_DEV)
            recv_c = lax.rem(me - s - 1 + 2 * N_DEV, N_DEV)
            rdma = pltpu.make_async_remote_copy(
                src_ref=out_ref.at[pl.ds(send_c * CHUNK, CHUNK), :],
                dst_ref=rs_scratch.at[s],
                send_sem=send_sem,
                recv_sem=rs_recv_sems.at[s],
                device_id=(right,),
                device_id_type=pl.DeviceIdType.MESH,
            )
            rdma.start()
            rdma.wait()
            out_ref[pl.ds(recv_c * CHUNK, CHUNK), :] = (
                out_ref[pl.ds(recv_c * CHUNK, CHUNK), :] + rs_scratch[s]
            )

        for t in range(N_DEV - 1):
            sc = lax.rem(me + 1 - t + 2 * N_DEV, N_DEV)
            rdma = pltpu.make_async_remote_copy(
                src_ref=out_ref.at[pl.ds(sc * CHUNK, CHUNK), :],
                dst_ref=out_ref.at[pl.ds(sc * CHUNK, CHUNK), :],
                send_sem=send_sem,
                recv_sem=ag_recv_sems.at[t],
                device_id=(right,),
                device_id_type=pl.DeviceIdType.MESH,
            )
            rdma.start()
            rdma.wait()

    return pl.pallas_call(
        body,
        out_shape=jax.ShapeDtypeStruct((rows, cols), jnp.float32),
        in_specs=[pl.BlockSpec(memory_space=pltpu.VMEM)],
        out_specs=pl.BlockSpec(memory_space=pltpu.VMEM),
        scratch_shapes=[
            pltpu.VMEM((N_DEV - 1, CHUNK, cols), jnp.float32),
            pltpu.SemaphoreType.DMA,
            pltpu.SemaphoreType.DMA((N_DEV - 1,)),
            pltpu.SemaphoreType.DMA((N_DEV - 1,)),
        ],
        compiler_params=pltpu.CompilerParams(collective_id=0),
    )(partial)


def kernel(x, Wq, K_ext, V_ext, Wo):
    me = lax.axis_index("i")

    xb = x[0].astype(jnp.bfloat16)
    q = jnp.dot(xb, Wq.astype(jnp.bfloat16),
                preferred_element_type=jnp.float32)
    q = q.reshape(SQ, H_PER, DH).astype(jnp.bfloat16)

    k = lax.dynamic_slice_in_dim(K_ext[0], me * H_PER, H_PER, axis=1)
    v = lax.dynamic_slice_in_dim(V_ext[0], me * H_PER, H_PER, axis=1)
    k = k.astype(jnp.bfloat16)
    v = v.astype(jnp.bfloat16)

    qi = jnp.arange(SQ)[:, None]
    ki = jnp.arange(SKV)[None, :]
    mask = (jnp.abs(qi - ki) <= 128) | (ki < 32) | (qi < 32)

    scores = jnp.einsum("ihd,jhd->hij", q, k,
                        preferred_element_type=jnp.float32) * SCALE
    scores = jnp.where(mask[None, :, :], scores, -1e9)
    scores_max = scores.max(axis=-1, keepdims=True)
    w = jnp.exp(scores - scores_max)
    w = w / w.sum(axis=-1, keepdims=True)

    ctx = jnp.einsum("hij,jhd->ihd", w.astype(jnp.bfloat16), v,
                     preferred_element_type=jnp.float32)
    ctx = ctx.reshape(SQ, H_PER * DH).astype(jnp.bfloat16)

    partial = jnp.dot(ctx, Wo.astype(jnp.bfloat16),
                      preferred_element_type=jnp.float32)

    out = _ring_allreduce(partial)
    return out[None]


# device time: 241556 ns/iter; 1.9509x vs baseline; 1.9509x over previous
import jax
import jax.numpy as jnp
from jax import lax
from jax.experimental import pallas as pl
from jax.experimental.pallas import tpu as pltpu

N_DEV = 16
SQ = 2048
SKV = 2048
D_MODEL = 1024
H_PER = 8
DH = 128
SCALE = 0.08838834764831843
QB = 128
N_BLK = SQ // QB
CHUNK = SQ // N_DEV
HALF = D_MODEL // 2


def _attn_compute(xb, Wq, k, v, Wo):

    def body(x_ref, wq_ref, k_ref, v_ref, wo_ref, out_ref):
        qb = pl.program_id(0)
        q = jnp.dot(x_ref[...], wq_ref[...],
                    preferred_element_type=jnp.float32)
        q = q.astype(jnp.bfloat16)

        row = lax.broadcasted_iota(jnp.int32, (QB, 1), 0)
        qi = qb * QB + row

        def out_proj(ctx_f32):
            out_ref[...] = jnp.dot(ctx_f32.astype(jnp.bfloat16), wo_ref[...],
                                   preferred_element_type=jnp.float32)

        @pl.when(qb == 0)
        def _dense():
            ki = lax.broadcasted_iota(jnp.int32, (QB, SKV), 1)
            mask = (jnp.abs(qi - ki) <= 128) | (ki < 32) | (qi < 32)
            ctxs = []
            for h in range(H_PER):
                q_h = q[:, h * DH:(h + 1) * DH]
                s = lax.dot_general(
                    q_h, k_ref[h],
                    dimension_numbers=(((1,), (1,)), ((), ())),
                    preferred_element_type=jnp.float32,
                ) * SCALE
                s = jnp.where(mask, s, -1e9)
                m = jnp.max(s, axis=-1, keepdims=True)
                w = jnp.exp(s - m)
                w = w / jnp.sum(w, axis=-1, keepdims=True)
                ctxs.append(jnp.dot(w.astype(jnp.bfloat16), v_ref[h],
                                    preferred_element_type=jnp.float32))
            out_proj(jnp.concatenate(ctxs, axis=1))

        @pl.when(qb > 0)
        def _sparse():
            kbs = [qb - qb, qb - 1, qb, jnp.minimum(qb + 1, N_BLK - 1)]
            valid = [qb >= 0, qb != 1, qb >= 0, qb != N_BLK - 1]
            col = lax.broadcasted_iota(jnp.int32, (QB, QB), 1)
            masks = []
            for kb, ok in zip(kbs, valid):
                ki = kb * QB + col
                mk = (jnp.abs(qi - ki) <= 128) | (ki < 32) | (qi < 32)
                masks.append(mk & ok)
            mask = jnp.concatenate(masks, axis=1)

            ctxs = []
            for h in range(H_PER):
                q_h = q[:, h * DH:(h + 1) * DH]
                s_slots = []
                v_slots = []
                for kb in kbs:
                    k_blk = k_ref[h, pl.ds(kb * QB, QB), :]
                    s_slots.append(lax.dot_general(
                        q_h, k_blk,
                        dimension_numbers=(((1,), (1,)), ((), ())),
                        preferred_element_type=jnp.float32,
                    ))
                    v_slots.append(v_ref[h, pl.ds(kb * QB, QB), :])
                s = jnp.concatenate(s_slots, axis=1) * SCALE
                s = jnp.where(mask, s, -1e9)
                m = jnp.max(s, axis=-1, keepdims=True)
                w = jnp.exp(s - m)
                w = w / jnp.sum(w, axis=-1, keepdims=True)
                v_sel = jnp.concatenate(v_slots, axis=0)
                ctxs.append(jnp.dot(w.astype(jnp.bfloat16), v_sel,
                                    preferred_element_type=jnp.float32))
            out_proj(jnp.concatenate(ctxs, axis=1))

    return pl.pallas_call(
        body,
        grid=(N_BLK,),
        out_shape=jax.ShapeDtypeStruct((SQ, D_MODEL), jnp.float32),
        in_specs=[
            pl.BlockSpec((QB, D_MODEL), lambda i: (i, 0)),
            pl.BlockSpec((D_MODEL, D_MODEL), lambda i: (0, 0)),
            pl.BlockSpec((H_PER, SKV, DH), lambda i: (0, 0, 0)),
            pl.BlockSpec((H_PER, SKV, DH), lambda i: (0, 0, 0)),
            pl.BlockSpec((D_MODEL, D_MODEL), lambda i: (0, 0)),
        ],
        out_specs=pl.BlockSpec((QB, D_MODEL), lambda i: (i, 0)),
    )(xb, Wq, k, v, Wo)


def _ring_allreduce(partial):

    def body(p_ref, out_ref,
             sndA, sndB, rsA, rsB, agA, agB,
             send_semA, send_semB, rsA_sems, rsB_sems, agA_sems, agB_sems):
        me = lax.axis_index("i")
        right = lax.rem(me + 1, N_DEV)
        left = lax.rem(me + N_DEV - 1, N_DEV)

        barrier_sem = pltpu.get_barrier_semaphore()
        for nbr in (left, right):
            pl.semaphore_signal(
                barrier_sem, inc=1,
                device_id=(nbr,), device_id_type=pl.DeviceIdType.MESH,
            )
        pl.semaphore_wait(barrier_sem, 2)

        out_ref[...] = p_ref[...]

        def rows(c):
            return pl.ds(c * CHUNK, CHUNK)

        for s in range(N_DEV - 1):
            cA = lax.rem(me - s + 2 * N_DEV, N_DEV)
            cB = lax.rem(me + s, N_DEV)
            sndA[...] = out_ref[rows(cA), :HALF].astype(jnp.bfloat16)
            sndB[...] = out_ref[rows(cB), HALF:].astype(jnp.bfloat16)
            rdmaA = pltpu.make_async_remote_copy(
                src_ref=sndA, dst_ref=rsA.at[s],
                send_sem=send_semA, recv_sem=rsA_sems.at[s],
                device_id=(right,), device_id_type=pl.DeviceIdType.MESH,
            )
            rdmaB = pltpu.make_async_remote_copy(
                src_ref=sndB, dst_ref=rsB.at[s],
                send_sem=send_semB, recv_sem=rsB_sems.at[s],
                device_id=(left,), device_id_type=pl.DeviceIdType.MESH,
            )
            rdmaA.start()
            rdmaB.start()
            rdmaA.wait()
            rdmaB.wait()
            rA = lax.rem(me - s - 1 + 2 * N_DEV, N_DEV)
            rB = lax.rem(me + s + 1, N_DEV)
            out_ref[rows(rA), :HALF] = (
                out_ref[rows(rA), :HALF] + rsA[s].astype(jnp.float32))
            out_ref[rows(rB), HALF:] = (
                out_ref[rows(rB), HALF:] + rsB[s].astype(jnp.float32))

        ownA = lax.rem(me + 1, N_DEV)
        ownB = lax.rem(me + N_DEV - 1, N_DEV)
        agA[N_DEV - 1] = out_ref[rows(ownA), :HALF].astype(jnp.bfloat16)
        agB[N_DEV - 1] = out_ref[rows(ownB), HALF:].astype(jnp.bfloat16)
        for t in range(N_DEV - 1):
            srcA = agA.at[N_DEV - 1] if t == 0 else agA.at[t - 1]
            srcB = agB.at[N_DEV - 1] if t == 0 else agB.at[t - 1]
            rdmaA = pltpu.make_async_remote_copy(
                src_ref=srcA, dst_ref=agA.at[t],
                send_sem=send_semA, recv_sem=agA_sems.at[t],
                device_id=(right,), device_id_type=pl.DeviceIdType.MESH,
            )
            rdmaB = pltpu.make_async_remote_copy(
                src_ref=srcB, dst_ref=agB.at[t],
                send_sem=send_semB, recv_sem=agB_sems.at[t],
                device_id=(left,), device_id_type=pl.DeviceIdType.MESH,
            )
            rdmaA.start()
            rdmaB.start()
            rdmaA.wait()
            rdmaB.wait()
            rcA = lax.rem(me - t + 2 * N_DEV, N_DEV)
            rcB = lax.rem(me + t, N_DEV)
            out_ref[rows(rcA), :HALF] = agA[t].astype(jnp.float32)
            out_ref[rows(rcB), HALF:] = agB[t].astype(jnp.float32)

    return pl.pallas_call(
        body,
        out_shape=jax.ShapeDtypeStruct((SQ, D_MODEL), jnp.float32),
        in_specs=[pl.BlockSpec(memory_space=pltpu.VMEM)],
        out_specs=pl.BlockSpec(memory_space=pltpu.VMEM),
        scratch_shapes=[
            pltpu.VMEM((CHUNK, HALF), jnp.bfloat16),
            pltpu.VMEM((CHUNK, HALF), jnp.bfloat16),
            pltpu.VMEM((N_DEV - 1, CHUNK, HALF), jnp.bfloat16),
            pltpu.VMEM((N_DEV - 1, CHUNK, HALF), jnp.bfloat16),
            pltpu.VMEM((N_DEV, CHUNK, HALF), jnp.bfloat16),
            pltpu.VMEM((N_DEV, CHUNK, HALF), jnp.bfloat16),
            pltpu.SemaphoreType.DMA,
            pltpu.SemaphoreType.DMA,
            pltpu.SemaphoreType.DMA((N_DEV - 1,)),
            pltpu.SemaphoreType.DMA((N_DEV - 1,)),
            pltpu.SemaphoreType.DMA((N_DEV - 1,)),
            pltpu.SemaphoreType.DMA((N_DEV - 1,)),
        ],
        compiler_params=pltpu.CompilerParams(collective_id=0),
    )(partial)


def kernel(x, Wq, K_ext, V_ext, Wo):
    me = lax.axis_index("i")

    xb = x[0].astype(jnp.bfloat16)
    k = lax.dynamic_slice_in_dim(K_ext[0], me * H_PER, H_PER, axis=1)
    v = lax.dynamic_slice_in_dim(V_ext[0], me * H_PER, H_PER, axis=1)
    k = k.transpose(1, 0, 2).astype(jnp.bfloat16)
    v = v.transpose(1, 0, 2).astype(jnp.bfloat16)

    partial = _attn_compute(xb, Wq.astype(jnp.bfloat16), k, v,
                            Wo.astype(jnp.bfloat16))
    out = _ring_allreduce(partial)
    return out[None]


# device time: 155354 ns/iter; 3.0334x vs baseline; 1.5549x over previous
import jax
import jax.numpy as jnp
from jax import lax
from jax.experimental import pallas as pl
from jax.experimental.pallas import tpu as pltpu

N_DEV = 16
SQ = 2048
SKV = 2048
D_MODEL = 1024
H_PER = 8
DH = 128
SCALE = 0.08838834764831843
QB = 128
N_BLK = SQ // QB
HALF = D_MODEL // 2
PC = SQ // 4
ZC = PC // 4
NEG = -1e9


def _attn_compute(xb, Wq, k, v, Wo):

    def body(x_ref, wq_ref, k_ref, v_ref, wo_ref, out_ref):
        qb = pl.program_id(0)
        qf = jnp.dot(x_ref[...], wq_ref[...],
                     preferred_element_type=jnp.float32)
        q = (qf * SCALE).astype(jnp.bfloat16)

        row = lax.broadcasted_iota(jnp.int32, (QB, 1), 0)
        qi = qb * QB + row

        def out_proj(ctx_f32):
            out_ref[...] = jnp.dot(ctx_f32.astype(jnp.bfloat16), wo_ref[...],
                                   preferred_element_type=jnp.float32)

        @pl.when(qb == 0)
        def _dense():
            ki = lax.broadcasted_iota(jnp.int32, (QB, SKV), 1)
            keep = (jnp.abs(qi - ki) <= 128) | (ki < 32) | (qi < 32)
            bias = jnp.where(keep, 0.0, NEG).astype(jnp.float32)
            ctxs = []
            for h in range(H_PER):
                q_h = q[:, h * DH:(h + 1) * DH]
                s = lax.dot_general(
                    q_h, k_ref[h],
                    dimension_numbers=(((1,), (1,)), ((), ())),
                    preferred_element_type=jnp.float32,
                ) + bias
                e = jnp.exp(s)
                r = 1.0 / jnp.sum(e, axis=-1, keepdims=True)
                ctx = jnp.dot(e.astype(jnp.bfloat16), v_ref[h],
                              preferred_element_type=jnp.float32)
                ctxs.append(ctx * r)
            out_proj(jnp.concatenate(ctxs, axis=1))

        @pl.when(qb > 0)
        def _sparse():
            kbs = [qb - qb, qb - 1, qb, jnp.minimum(qb + 1, N_BLK - 1)]
            valid = [qb >= 0, qb != 1, qb >= 0, qb != N_BLK - 1]
            col = lax.broadcasted_iota(jnp.int32, (QB, QB), 1)
            biases = []
            for kb, ok in zip(kbs, valid):
                ki = kb * QB + col
                keep = ((jnp.abs(qi - ki) <= 128) | (ki < 32) | (qi < 32)) & ok
                biases.append(jnp.where(keep, 0.0, NEG))
            bias = jnp.concatenate(biases, axis=1).astype(jnp.float32)

            ctxs = []
            for h in range(H_PER):
                q_h = q[:, h * DH:(h + 1) * DH]
                s_slots = []
                v_slots = []
                for kb in kbs:
                    k_blk = k_ref[h, pl.ds(kb * QB, QB), :]
                    s_slots.append(lax.dot_general(
                        q_h, k_blk,
                        dimension_numbers=(((1,), (1,)), ((), ())),
                        preferred_element_type=jnp.float32,
                    ))
                    v_slots.append(v_ref[h, pl.ds(kb * QB, QB), :])
                s = jnp.concatenate(s_slots, axis=1) + bias
                e = jnp.exp(s)
                r = 1.0 / jnp.sum(e, axis=-1, keepdims=True)
                v_sel = jnp.concatenate(v_slots, axis=0)
                ctx = jnp.dot(e.astype(jnp.bfloat16), v_sel,
                              preferred_element_type=jnp.float32)
                ctxs.append(ctx * r)
            out_proj(jnp.concatenate(ctxs, axis=1))

    return pl.pallas_call(
        body,
        grid=(N_BLK,),
        out_shape=jax.ShapeDtypeStruct((SQ, D_MODEL), jnp.float32),
        in_specs=[
            pl.BlockSpec((QB, D_MODEL), lambda i: (i, 0)),
            pl.BlockSpec((D_MODEL, D_MODEL), lambda i: (0, 0)),
            pl.BlockSpec((H_PER, SKV, DH), lambda i: (0, 0, 0)),
            pl.BlockSpec((H_PER, SKV, DH), lambda i: (0, 0, 0)),
            pl.BlockSpec((D_MODEL, D_MODEL), lambda i: (0, 0)),
        ],
        out_specs=pl.BlockSpec((QB, D_MODEL), lambda i: (i, 0)),
    )(xb, Wq, k, v, Wo)


def _ring_allreduce(partial):

    def body(p_ref, out_ref,
             snd1A, snd1B, snd2A, snd2B, p1A, p1B, p2A, p2B,
             g2A, g2B, g1A, g1B,
             send_semA, send_semB,
             p1A_s, p1B_s, p2A_s, p2B_s, g2A_s, g2B_s, g1A_s, g1B_s):
        me = lax.axis_index("i")
        pin = lax.rem(me, 4)
        zi = me // 4
        base = me - pin

        def plane_dev(p):
            return base + lax.rem(p + 8, 4)

        def z_dev(z):
            return lax.rem(z + 8, 4) * 4 + pin

        nbrs = [plane_dev(pin + 1), plane_dev(pin - 1),
                z_dev(zi + 1), z_dev(zi - 1)]
        barrier_sem = pltpu.get_barrier_semaphore()
        for nbr in nbrs:
            pl.semaphore_signal(
                barrier_sem, inc=1,
                device_id=(nbr,), device_id_type=pl.DeviceIdType.MESH,
            )
        pl.semaphore_wait(barrier_sem, 4)

        out_ref[...] = p_ref[...]

        def send_pair(srcA, dstA, semA, devA, srcB, dstB, semB, devB):
            rdmaA = pltpu.make_async_remote_copy(
                src_ref=srcA, dst_ref=dstA, send_sem=send_semA,
                recv_sem=semA, device_id=(devA,),
                device_id_type=pl.DeviceIdType.MESH,
            )
            rdmaB = pltpu.make_async_remote_copy(
                src_ref=srcB, dst_ref=dstB, send_sem=send_semB,
                recv_sem=semB, device_id=(devB,),
                device_id_type=pl.DeviceIdType.MESH,
            )
            rdmaA.start()
            rdmaB.start()
            rdmaA.wait()
            rdmaB.wait()

        rightP = plane_dev(pin + 1)
        leftP = plane_dev(pin - 1)
        upZ = z_dev(zi + 1)
        downZ = z_dev(zi - 1)

        for s in range(3):
            cA = lax.rem(pin - s + 8, 4)
            cB = lax.rem(pin + s, 4)
            snd1A[...] = out_ref[pl.ds(cA * PC, PC), :HALF].astype(jnp.bfloat16)
            snd1B[...] = out_ref[pl.ds(cB * PC, PC), HALF:].astype(jnp.bfloat16)
            send_pair(snd1A, p1A.at[s], p1A_s.at[s], rightP,
                      snd1B, p1B.at[s], p1B_s.at[s], leftP)
            rA = lax.rem(pin - s - 1 + 8, 4)
            rB = lax.rem(pin + s + 1, 4)
            out_ref[pl.ds(rA * PC, PC), :HALF] = (
                out_ref[pl.ds(rA * PC, PC), :HALF] + p1A[s].astype(jnp.float32))
            out_ref[pl.ds(rB * PC, PC), HALF:] = (
                out_ref[pl.ds(rB * PC, PC), HALF:] + p1B[s].astype(jnp.float32))

        qA = lax.rem(pin + 1, 4) * PC
        qB = lax.rem(pin + 3, 4) * PC

        for s in range(3):
            cA = lax.rem(zi - s + 8, 4)
            cB = lax.rem(zi + s, 4)
            snd2A[...] = out_ref[pl.ds(qA + cA * ZC, ZC),
                                 :HALF].astype(jnp.bfloat16)
            snd2B[...] = out_ref[pl.ds(qB + cB * ZC, ZC),
                                 HALF:].astype(jnp.bfloat16)
            send_pair(snd2A, p2A.at[s], p2A_s.at[s], upZ,
                      snd2B, p2B.at[s], p2B_s.at[s], downZ)
            rA = lax.rem(zi - s - 1 + 8, 4)
            rB = lax.rem(zi + s + 1, 4)
            out_ref[pl.ds(qA + rA * ZC, ZC), :HALF] = (
                out_ref[pl.ds(qA + rA * ZC, ZC), :HALF]
                + p2A[s].astype(jnp.float32))
            out_ref[pl.ds(qB + rB * ZC, ZC), HALF:] = (
                out_ref[pl.ds(qB + rB * ZC, ZC), HALF:]
                + p2B[s].astype(jnp.float32))

        ozA = lax.rem(zi + 1, 4)
        ozB = lax.rem(zi + 3, 4)

        g2A[3] = out_ref[pl.ds(qA + ozA * ZC, ZC), :HALF].astype(jnp.bfloat16)
        g2B[3] = out_ref[pl.ds(qB + ozB * ZC, ZC), HALF:].astype(jnp.bfloat16)
        for t in range(3):
            srcA = g2A.at[3 if t == 0 else t - 1]
            srcB = g2B.at[3 if t == 0 else t - 1]
            send_pair(srcA, g2A.at[t], g2A_s.at[t], upZ,
                      srcB, g2B.at[t], g2B_s.at[t], downZ)
            rcA = lax.rem(zi - t + 8, 4)
            rcB = lax.rem(zi + t, 4)
            out_ref[pl.ds(qA + rcA * ZC, ZC), :HALF] = (
                g2A[t].astype(jnp.float32))
            out_ref[pl.ds(qB + rcB * ZC, ZC), HALF:] = (
                g2B[t].astype(jnp.float32))

        g1A[3] = out_ref[pl.ds(qA, PC), :HALF].astype(jnp.bfloat16)
        g1B[3] = out_ref[pl.ds(qB, PC), HALF:].astype(jnp.bfloat16)
        for t in range(3):
            srcA = g1A.at[3 if t == 0 else t - 1]
            srcB = g1B.at[3 if t == 0 else t - 1]
            send_pair(srcA, g1A.at[t], g1A_s.at[t], rightP,
                      srcB, g1B.at[t], g1B_s.at[t], leftP)
            rcA = lax.rem(pin + 1 - t - 1 + 8, 4) * PC
            rcB = lax.rem(pin + 3 + t + 1, 4) * PC
            out_ref[pl.ds(rcA, PC), :HALF] = g1A[t].astype(jnp.float32)
            out_ref[pl.ds(rcB, PC), HALF:] = g1B[t].astype(jnp.float32)

    return pl.pallas_call(
        body,
        out_shape=jax.ShapeDtypeStruct((SQ, D_MODEL), jnp.float32),
        in_specs=[pl.BlockSpec(memory_space=pltpu.VMEM)],
        out_specs=pl.BlockSpec(memory_space=pltpu.VMEM),
        scratch_shapes=[
            pltpu.VMEM((PC, HALF), jnp.bfloat16),
            pltpu.VMEM((PC, HALF), jnp.bfloat16),
            pltpu.VMEM((ZC, HALF), jnp.bfloat16),
            pltpu.VMEM((ZC, HALF), jnp.bfloat16),
            pltpu.VMEM((3, PC, HALF), jnp.bfloat16),
            pltpu.VMEM((3, PC, HALF), jnp.bfloat16),
            pltpu.VMEM((3, ZC, HALF), jnp.bfloat16),
            pltpu.VMEM((3, ZC, HALF), jnp.bfloat16),
            pltpu.VMEM((4, ZC, HALF), jnp.bfloat16),
            pltpu.VMEM((4, ZC, HALF), jnp.bfloat16),
            pltpu.VMEM((4, PC, HALF), jnp.bfloat16),
            pltpu.VMEM((4, PC, HALF), jnp.bfloat16),
            pltpu.SemaphoreType.DMA,
            pltpu.SemaphoreType.DMA,
            pltpu.SemaphoreType.DMA((3,)),
            pltpu.SemaphoreType.DMA((3,)),
            pltpu.SemaphoreType.DMA((3,)),
            pltpu.SemaphoreType.DMA((3,)),
            pltpu.SemaphoreType.DMA((3,)),
            pltpu.SemaphoreType.DMA((3,)),
            pltpu.SemaphoreType.DMA((3,)),
            pltpu.SemaphoreType.DMA((3,)),
        ],
        compiler_params=pltpu.CompilerParams(collective_id=0),
    )(partial)


def kernel(x, Wq, K_ext, V_ext, Wo):
    me = lax.axis_index("i")

    xb = x[0].astype(jnp.bfloat16)
    k = lax.dynamic_slice_in_dim(K_ext[0], me * H_PER, H_PER, axis=1)
    v = lax.dynamic_slice_in_dim(V_ext[0], me * H_PER, H_PER, axis=1)
    k = k.transpose(1, 0, 2).astype(jnp.bfloat16)
    v = v.transpose(1, 0, 2).astype(jnp.bfloat16)

    partial = _attn_compute(xb, Wq.astype(jnp.bfloat16), k, v,
                            Wo.astype(jnp.bfloat16))
    out = _ring_allreduce(partial)
    return out[None]


# device time: 155134 ns/iter; 3.0377x vs baseline; 1.0014x over previous
import jax
import jax.numpy as jnp
from jax import lax
from jax.experimental import pallas as pl
from jax.experimental.pallas import tpu as pltpu

N_DEV = 16
SQ = 2048
SKV = 2048
D_MODEL = 1024
H_PER = 8
DH = 128
SCALE = 0.08838834764831843
QB = 128
N_BLK = SQ // QB
HALF = D_MODEL // 2
PC = SQ // 4
ZC = PC // 4
NEG = -1e9


def _attn_compute(xb, Wq, k, v, Wo):

    def body(x_ref, wq_ref, k_ref, v_ref, wo_ref, out_ref):
        qb = pl.program_id(0)
        qf = jnp.dot(x_ref[...], wq_ref[...],
                     preferred_element_type=jnp.float32)
        q = (qf * SCALE).astype(jnp.bfloat16)

        row = lax.broadcasted_iota(jnp.int32, (QB, 1), 0)
        qi = qb * QB + row

        def out_proj(ctx_f32):
            out_ref[...] = jnp.dot(ctx_f32.astype(jnp.bfloat16), wo_ref[...],
                                   preferred_element_type=jnp.float32)

        @pl.when(qb == 0)
        def _dense():
            ki = lax.broadcasted_iota(jnp.int32, (QB, SKV), 1)
            keep = (jnp.abs(qi - ki) <= 128) | (ki < 32) | (qi < 32)
            bias = jnp.where(keep, 0.0, NEG).astype(jnp.bfloat16)
            ctxs = []
            for h in range(H_PER):
                q_h = q[:, h * DH:(h + 1) * DH]
                s = lax.dot_general(
                    q_h, k_ref[:, h * DH:(h + 1) * DH],
                    dimension_numbers=(((1,), (1,)), ((), ())),
                    preferred_element_type=jnp.float32,
                ).astype(jnp.bfloat16) + bias
                e = jnp.exp(s)
                r = 1.0 / jnp.sum(e, axis=-1, keepdims=True,
                                  dtype=jnp.float32)
                ctx = jnp.dot(e, v_ref[:, h * DH:(h + 1) * DH],
                              preferred_element_type=jnp.float32)
                ctxs.append(ctx * r)
            out_proj(jnp.concatenate(ctxs, axis=1))

        @pl.when(qb > 0)
        def _sparse():
            kbs = [qb - qb, qb - 1, qb, jnp.minimum(qb + 1, N_BLK - 1)]
            valid = [qb >= 0, qb != 1, qb >= 0, qb != N_BLK - 1]
            col = lax.broadcasted_iota(jnp.int32, (QB, QB), 1)
            biases = []
            for kb, ok in zip(kbs, valid):
                ki = kb * QB + col
                keep = ((jnp.abs(qi - ki) <= 128) | (ki < 32) | (qi < 32)) & ok
                biases.append(jnp.where(keep, 0.0, NEG))
            bias = jnp.concatenate(biases, axis=1).astype(jnp.bfloat16)

            ctxs = []
            for h in range(H_PER):
                q_h = q[:, h * DH:(h + 1) * DH]
                s_slots = []
                v_slots = []
                for kb in kbs:
                    k_blk = k_ref[pl.ds(kb * QB, QB), h * DH:(h + 1) * DH]
                    s_slots.append(lax.dot_general(
                        q_h, k_blk,
                        dimension_numbers=(((1,), (1,)), ((), ())),
                        preferred_element_type=jnp.float32,
                    ).astype(jnp.bfloat16))
                    v_slots.append(v_ref[pl.ds(kb * QB, QB),
                                         h * DH:(h + 1) * DH])
                s = jnp.concatenate(s_slots, axis=1) + bias
                e = jnp.exp(s)
                r = 1.0 / jnp.sum(e, axis=-1, keepdims=True,
                                  dtype=jnp.float32)
                v_sel = jnp.concatenate(v_slots, axis=0)
                ctx = jnp.dot(e, v_sel, preferred_element_type=jnp.float32)
                ctxs.append(ctx * r)
            out_proj(jnp.concatenate(ctxs, axis=1))

    return pl.pallas_call(
        body,
        grid=(N_BLK,),
        out_shape=jax.ShapeDtypeStruct((SQ, D_MODEL), jnp.float32),
        in_specs=[
            pl.BlockSpec((QB, D_MODEL), lambda i: (i, 0)),
            pl.BlockSpec((D_MODEL, D_MODEL), lambda i: (0, 0)),
            pl.BlockSpec((SKV, H_PER * DH), lambda i: (0, 0)),
            pl.BlockSpec((SKV, H_PER * DH), lambda i: (0, 0)),
            pl.BlockSpec((D_MODEL, D_MODEL), lambda i: (0, 0)),
        ],
        out_specs=pl.BlockSpec((QB, D_MODEL), lambda i: (i, 0)),
    )(xb, Wq, k, v, Wo)


def _ring_allreduce(partial):

    def body(p_ref, out_ref,
             snd1A, snd1B, snd2A, snd2B, p1A, p1B, p2A, p2B,
             g2A, g2B, g1A, g1B,
             send_semA, send_semB,
             p1A_s, p1B_s, p2A_s, p2B_s, g2A_s, g2B_s, g1A_s, g1B_s):
        me = lax.axis_index("i")
        pin = lax.rem(me, 4)
        zi = me // 4
        base = me - pin

        def plane_dev(p):
            return base + lax.rem(p + 8, 4)

        def z_dev(z):
            return lax.rem(z + 8, 4) * 4 + pin

        nbrs = [plane_dev(pin + 1), plane_dev(pin - 1),
                z_dev(zi + 1), z_dev(zi - 1)]
        barrier_sem = pltpu.get_barrier_semaphore()
        for nbr in nbrs:
            pl.semaphore_signal(
                barrier_sem, inc=1,
                device_id=(nbr,), device_id_type=pl.DeviceIdType.MESH,
            )
        pl.semaphore_wait(barrier_sem, 4)

        out_ref[...] = p_ref[...]

        def send_pair(srcA, dstA, semA, devA, srcB, dstB, semB, devB,
                      stageA=None, stageB=None):
            if stageA is not None:
                stageA()
            rdmaA = pltpu.make_async_remote_copy(
                src_ref=srcA, dst_ref=dstA, send_sem=send_semA,
                recv_sem=semA, device_id=(devA,),
                device_id_type=pl.DeviceIdType.MESH,
            )
            rdmaA.start()
            if stageB is not None:
                stageB()
            rdmaB = pltpu.make_async_remote_copy(
                src_ref=srcB, dst_ref=dstB, send_sem=send_semB,
                recv_sem=semB, device_id=(devB,),
                device_id_type=pl.DeviceIdType.MESH,
            )
            rdmaB.start()
            rdmaA.wait()
            rdmaB.wait()

        rightP = plane_dev(pin + 1)
        leftP = plane_dev(pin - 1)
        upZ = z_dev(zi + 1)
        downZ = z_dev(zi - 1)

        for s in range(3):
            cA = lax.rem(pin - s + 8, 4)
            cB = lax.rem(pin + s, 4)

            def _stA(cA=cA):
                snd1A[...] = out_ref[pl.ds(cA * PC, PC),
                                     :HALF].astype(jnp.bfloat16)

            def _stB(cB=cB):
                snd1B[...] = out_ref[pl.ds(cB * PC, PC),
                                     HALF:].astype(jnp.bfloat16)

            send_pair(snd1A, p1A.at[s], p1A_s.at[s], rightP,
                      snd1B, p1B.at[s], p1B_s.at[s], leftP,
                      stageA=_stA, stageB=_stB)
            rA = lax.rem(pin - s - 1 + 8, 4)
            rB = lax.rem(pin + s + 1, 4)
            out_ref[pl.ds(rA * PC, PC), :HALF] = (
                out_ref[pl.ds(rA * PC, PC), :HALF] + p1A[s].astype(jnp.float32))
            out_ref[pl.ds(rB * PC, PC), HALF:] = (
                out_ref[pl.ds(rB * PC, PC), HALF:] + p1B[s].astype(jnp.float32))

        qA = lax.rem(pin + 1, 4) * PC
        qB = lax.rem(pin + 3, 4) * PC

        for s in range(3):
            cA = lax.rem(zi - s + 8, 4)
            cB = lax.rem(zi + s, 4)

            def _stA(cA=cA):
                snd2A[...] = out_ref[pl.ds(qA + cA * ZC, ZC),
                                     :HALF].astype(jnp.bfloat16)

            def _stB(cB=cB):
                snd2B[...] = out_ref[pl.ds(qB + cB * ZC, ZC),
                                     HALF:].astype(jnp.bfloat16)

            send_pair(snd2A, p2A.at[s], p2A_s.at[s], upZ,
                      snd2B, p2B.at[s], p2B_s.at[s], downZ,
                      stageA=_stA, stageB=_stB)
            rA = lax.rem(zi - s - 1 + 8, 4)
            rB = lax.rem(zi + s + 1, 4)
            out_ref[pl.ds(qA + rA * ZC, ZC), :HALF] = (
                out_ref[pl.ds(qA + rA * ZC, ZC), :HALF]
                + p2A[s].astype(jnp.float32))
            out_ref[pl.ds(qB + rB * ZC, ZC), HALF:] = (
                out_ref[pl.ds(qB + rB * ZC, ZC), HALF:]
                + p2B[s].astype(jnp.float32))

        ozA = lax.rem(zi + 1, 4)
        ozB = lax.rem(zi + 3, 4)

        g2A[3] = out_ref[pl.ds(qA + ozA * ZC, ZC), :HALF].astype(jnp.bfloat16)
        g2B[3] = out_ref[pl.ds(qB + ozB * ZC, ZC), HALF:].astype(jnp.bfloat16)
        for t in range(3):
            srcA = g2A.at[3 if t == 0 else t - 1]
            srcB = g2B.at[3 if t == 0 else t - 1]
            send_pair(srcA, g2A.at[t], g2A_s.at[t], upZ,
                      srcB, g2B.at[t], g2B_s.at[t], downZ)
            rcA = lax.rem(zi - t + 8, 4)
            rcB = lax.rem(zi + t, 4)
            out_ref[pl.ds(qA + rcA * ZC, ZC), :HALF] = (
                g2A[t].astype(jnp.float32))
            out_ref[pl.ds(qB + rcB * ZC, ZC), HALF:] = (
                g2B[t].astype(jnp.float32))

        g1A[3] = out_ref[pl.ds(qA, PC), :HALF].astype(jnp.bfloat16)
        g1B[3] = out_ref[pl.ds(qB, PC), HALF:].astype(jnp.bfloat16)
        for t in range(3):
            srcA = g1A.at[3 if t == 0 else t - 1]
            srcB = g1B.at[3 if t == 0 else t - 1]
            send_pair(srcA, g1A.at[t], g1A_s.at[t], rightP,
                      srcB, g1B.at[t], g1B_s.at[t], leftP)
            rcA = lax.rem(pin + 1 - t - 1 + 8, 4) * PC
            rcB = lax.rem(pin + 3 + t + 1, 4) * PC
            out_ref[pl.ds(rcA, PC), :HALF] = g1A[t].astype(jnp.float32)
            out_ref[pl.ds(rcB, PC), HALF:] = g1B[t].astype(jnp.float32)

    return pl.pallas_call(
        body,
        out_shape=jax.ShapeDtypeStruct((SQ, D_MODEL), jnp.float32),
        in_specs=[pl.BlockSpec(memory_space=pltpu.VMEM)],
        out_specs=pl.BlockSpec(memory_space=pltpu.VMEM),
        scratch_shapes=[
            pltpu.VMEM((PC, HALF), jnp.bfloat16),
            pltpu.VMEM((PC, HALF), jnp.bfloat16),
            pltpu.VMEM((ZC, HALF), jnp.bfloat16),
            pltpu.VMEM((ZC, HALF), jnp.bfloat16),
            pltpu.VMEM((3, PC, HALF), jnp.bfloat16),
            pltpu.VMEM((3, PC, HALF), jnp.bfloat16),
            pltpu.VMEM((3, ZC, HALF), jnp.bfloat16),
            pltpu.VMEM((3, ZC, HALF), jnp.bfloat16),
            pltpu.VMEM((4, ZC, HALF), jnp.bfloat16),
            pltpu.VMEM((4, ZC, HALF), jnp.bfloat16),
            pltpu.VMEM((4, PC, HALF), jnp.bfloat16),
            pltpu.VMEM((4, PC, HALF), jnp.bfloat16),
            pltpu.SemaphoreType.DMA,
            pltpu.SemaphoreType.DMA,
            pltpu.SemaphoreType.DMA((3,)),
            pltpu.SemaphoreType.DMA((3,)),
            pltpu.SemaphoreType.DMA((3,)),
            pltpu.SemaphoreType.DMA((3,)),
            pltpu.SemaphoreType.DMA((3,)),
            pltpu.SemaphoreType.DMA((3,)),
            pltpu.SemaphoreType.DMA((3,)),
            pltpu.SemaphoreType.DMA((3,)),
        ],
        compiler_params=pltpu.CompilerParams(collective_id=0),
    )(partial)


def kernel(x, Wq, K_ext, V_ext, Wo):
    me = lax.axis_index("i")

    xb = x[0].astype(jnp.bfloat16)
    k = lax.dynamic_slice_in_dim(K_ext[0], me * H_PER, H_PER, axis=1)
    v = lax.dynamic_slice_in_dim(V_ext[0], me * H_PER, H_PER, axis=1)
    k = k.reshape(SKV, H_PER * DH).astype(jnp.bfloat16)
    v = v.reshape(SKV, H_PER * DH).astype(jnp.bfloat16)

    partial = _attn_compute(xb, Wq.astype(jnp.bfloat16), k, v,
                            Wo.astype(jnp.bfloat16))
    out = _ring_allreduce(partial)
    return out[None]


# device time: 140512 ns/iter; 3.3538x vs baseline; 1.1041x over previous
import jax
import jax.numpy as jnp
from jax import lax
from jax.experimental import pallas as pl
from jax.experimental.pallas import tpu as pltpu

N_DEV = 16
SQ = 2048
SKV = 2048
D_MODEL = 1024
H_PER = 8
DH = 128
SCALE = 0.08838834764831843
QB = 128
N_BLK = SQ // QB
HALF = D_MODEL // 2
PC = SQ // 4
ZC = PC // 4
NEG = -1e9


def _fused(xb, Wq, k_hbm, v_hbm, Wo, me):
    def body(me_ref, x_ref, wq_ref, kh_ref, vh_ref, wo_ref, out_ref,
             kf, vf, kb, vb,
             snd1A, snd1B, snd2A, snd2B, p1A, p1B, p2A, p2B,
             g2A, g2B, g1A, g1B,
             kv_sems, send_semA, send_semB,
             p1A_s, p1B_s, p2A_s, p2B_s, g2A_s, g2B_s, g1A_s, g1B_s):
        me = me_ref[0]
        pin = lax.rem(me, 4)
        zi = me // 4
        base = me - pin

        def plane_dev(p):
            return base + lax.rem(p + 8, 4)

        def z_dev(z):
            return lax.rem(z + 8, 4) * 4 + pin

        nbrs = [plane_dev(pin + 1), plane_dev(pin - 1),
                z_dev(zi + 1), z_dev(zi - 1)]
        barrier_sem = pltpu.get_barrier_semaphore()
        for nbr in nbrs:
            pl.semaphore_signal(
                barrier_sem, inc=1,
                device_id=(nbr,), device_id_type=pl.DeviceIdType.MESH,
            )

        kcp = pltpu.make_async_copy(
            kh_ref.at[:, pl.ds(me * H_PER, H_PER), :], kf, kv_sems.at[0])
        vcp = pltpu.make_async_copy(
            vh_ref.at[:, pl.ds(me * H_PER, H_PER), :], vf, kv_sems.at[1])
        kcp.start()
        vcp.start()
        kcp.wait()
        vcp.wait()
        kb[...] = kf[...].reshape(SKV, H_PER * DH).astype(jnp.bfloat16)
        vb[...] = vf[...].reshape(SKV, H_PER * DH).astype(jnp.bfloat16)

        pl.semaphore_wait(barrier_sem, 4)

        def qproj(b):
            x_blk = x_ref[pl.ds(b * QB, QB), :]
            qf = jnp.dot(x_blk, wq_ref[...],
                         preferred_element_type=jnp.float32)
            return (qf * SCALE).astype(jnp.bfloat16)

        def out_proj(b, ctx_f32):
            out_ref[pl.ds(b * QB, QB), :] = jnp.dot(
                ctx_f32.astype(jnp.bfloat16), wo_ref[...],
                preferred_element_type=jnp.float32)

        def dense_block0():
            q = qproj(0)
            qi = lax.broadcasted_iota(jnp.int32, (QB, 1), 0)
            ki = lax.broadcasted_iota(jnp.int32, (QB, SKV), 1)
            keep = (jnp.abs(qi - ki) <= 128) | (ki < 32) | (qi < 32)
            bias = jnp.where(keep, 0.0, NEG).astype(jnp.float32)
            ctxs = []
            for h in range(H_PER):
                q_h = q[:, h * DH:(h + 1) * DH]
                s = lax.dot_general(
                    q_h, kb[:, h * DH:(h + 1) * DH],
                    dimension_numbers=(((1,), (1,)), ((), ())),
                    preferred_element_type=jnp.float32,
                ) + bias
                e = jnp.exp(s)
                r = 1.0 / jnp.sum(e, axis=-1, keepdims=True)
                ctx = jnp.dot(e.astype(jnp.bfloat16),
                              vb[:, h * DH:(h + 1) * DH],
                              preferred_element_type=jnp.float32)
                ctxs.append(ctx * r)
            out_proj(0, jnp.concatenate(ctxs, axis=1))

        def sparse_block(b):
            q = qproj(b)
            row = lax.broadcasted_iota(jnp.int32, (QB, 1), 0)
            qi = b * QB + row
            kbs = [b - b, jnp.maximum(b - 1, 0), b,
                   jnp.minimum(b + 1, N_BLK - 1)]
            valid = [b >= 0, b != 1, b >= 0, b != N_BLK - 1]
            col = lax.broadcasted_iota(jnp.int32, (QB, QB), 1)
            biases = []
            for kb_i, ok in zip(kbs, valid):
                ki = kb_i * QB + col
                keep = ((jnp.abs(qi - ki) <= 128) | (ki < 32)
                        | (qi < 32)) & ok
                biases.append(jnp.where(keep, 0.0, NEG))
            bias = jnp.concatenate(biases, axis=1).astype(jnp.float32)

            ctxs = []
            for h in range(H_PER):
                q_h = q[:, h * DH:(h + 1) * DH]
                s_slots = []
                v_slots = []
                for kb_i in kbs:
                    s_slots.append(lax.dot_general(
                        q_h, kb[pl.ds(kb_i * QB, QB), h * DH:(h + 1) * DH],
                        dimension_numbers=(((1,), (1,)), ((), ())),
                        preferred_element_type=jnp.float32,
                    ))
                    v_slots.append(vb[pl.ds(kb_i * QB, QB),
                                      h * DH:(h + 1) * DH])
                s = jnp.concatenate(s_slots, axis=1) + bias
                e = jnp.exp(s)
                r = 1.0 / jnp.sum(e, axis=-1, keepdims=True)
                v_sel = jnp.concatenate(v_slots, axis=0)
                ctx = jnp.dot(e.astype(jnp.bfloat16), v_sel,
                              preferred_element_type=jnp.float32)
                ctxs.append(ctx * r)
            out_proj(b, jnp.concatenate(ctxs, axis=1))

        def compute_quarter(qtr):
            for i in range(4):
                b = qtr * 4 + i

                @pl.when(b > 0)
                def _():
                    sparse_block(b)

        def send_pair(srcA, dstA, semA, devA, srcB, dstB, semB, devB,
                      stageA=None, stageB=None):
            if stageA is not None:
                stageA()
            rdmaA = pltpu.make_async_remote_copy(
                src_ref=srcA, dst_ref=dstA, send_sem=send_semA,
                recv_sem=semA, device_id=(devA,),
                device_id_type=pl.DeviceIdType.MESH,
            )
            rdmaA.start()
            if stageB is not None:
                stageB()
            rdmaB = pltpu.make_async_remote_copy(
                src_ref=srcB, dst_ref=dstB, send_sem=send_semB,
                recv_sem=semB, device_id=(devB,),
                device_id_type=pl.DeviceIdType.MESH,
            )
            rdmaB.start()
            return rdmaA, rdmaB

        rightP = plane_dev(pin + 1)
        leftP = plane_dev(pin - 1)
        upZ = z_dev(zi + 1)
        downZ = z_dev(zi - 1)

        dense_block0()
        compute_quarter(pin)
        for s in range(3):
            cA = lax.rem(pin - s + 8, 4)
            cB = lax.rem(pin + s, 4)

            def _stA(cA=cA):
                snd1A[...] = out_ref[pl.ds(cA * PC, PC),
                                     :HALF].astype(jnp.bfloat16)

            def _stB(cB=cB):
                snd1B[...] = out_ref[pl.ds(cB * PC, PC),
                                     HALF:].astype(jnp.bfloat16)

            rdmaA, rdmaB = send_pair(
                snd1A, p1A.at[s], p1A_s.at[s], rightP,
                snd1B, p1B.at[s], p1B_s.at[s], leftP,
                stageA=_stA, stageB=_stB)

            if s == 0:
                compute_quarter(lax.rem(pin + 3, 4))
                compute_quarter(lax.rem(pin + 1, 4))
            elif s == 1:
                compute_quarter(lax.rem(pin + 2, 4))

            rdmaA.wait()
            rdmaB.wait()
            rA = lax.rem(pin - s - 1 + 8, 4)
            rB = lax.rem(pin + s + 1, 4)
            out_ref[pl.ds(rA * PC, PC), :HALF] = (
                out_ref[pl.ds(rA * PC, PC), :HALF] + p1A[s].astype(jnp.float32))
            out_ref[pl.ds(rB * PC, PC), HALF:] = (
                out_ref[pl.ds(rB * PC, PC), HALF:] + p1B[s].astype(jnp.float32))

        qA = lax.rem(pin + 1, 4) * PC
        qB = lax.rem(pin + 3, 4) * PC

        for s in range(3):
            cA = lax.rem(zi - s + 8, 4)
            cB = lax.rem(zi + s, 4)

            def _stA(cA=cA):
                snd2A[...] = out_ref[pl.ds(qA + cA * ZC, ZC),
                                     :HALF].astype(jnp.bfloat16)

            def _stB(cB=cB):
                snd2B[...] = out_ref[pl.ds(qB + cB * ZC, ZC),
                                     HALF:].astype(jnp.bfloat16)

            rdmaA, rdmaB = send_pair(
                snd2A, p2A.at[s], p2A_s.at[s], upZ,
                snd2B, p2B.at[s], p2B_s.at[s], downZ,
                stageA=_stA, stageB=_stB)
            rdmaA.wait()
            rdmaB.wait()
            rA = lax.rem(zi - s - 1 + 8, 4)
            rB = lax.rem(zi + s + 1, 4)
            out_ref[pl.ds(qA + rA * ZC, ZC), :HALF] = (
                out_ref[pl.ds(qA + rA * ZC, ZC), :HALF]
                + p2A[s].astype(jnp.float32))
            out_ref[pl.ds(qB + rB * ZC, ZC), HALF:] = (
                out_ref[pl.ds(qB + rB * ZC, ZC), HALF:]
                + p2B[s].astype(jnp.float32))

        ozA = lax.rem(zi + 1, 4)
        ozB = lax.rem(zi + 3, 4)

        g2A[3] = out_ref[pl.ds(qA + ozA * ZC, ZC), :HALF].astype(jnp.bfloat16)
        g2B[3] = out_ref[pl.ds(qB + ozB * ZC, ZC), HALF:].astype(jnp.bfloat16)
        for t in range(3):
            srcA = g2A.at[3 if t == 0 else t - 1]
            srcB = g2B.at[3 if t == 0 else t - 1]
            rdmaA, rdmaB = send_pair(srcA, g2A.at[t], g2A_s.at[t], upZ,
                                     srcB, g2B.at[t], g2B_s.at[t], downZ)
            rdmaA.wait()
            rdmaB.wait()
            rcA = lax.rem(zi - t + 8, 4)
            rcB = lax.rem(zi + t, 4)
            out_ref[pl.ds(qA + rcA * ZC, ZC), :HALF] = (
                g2A[t].astype(jnp.float32))
            out_ref[pl.ds(qB + rcB * ZC, ZC), HALF:] = (
                g2B[t].astype(jnp.float32))

        g1A[3] = out_ref[pl.ds(qA, PC), :HALF].astype(jnp.bfloat16)
        g1B[3] = out_ref[pl.ds(qB, PC), HALF:].astype(jnp.bfloat16)
        for t in range(3):
            srcA = g1A.at[3 if t == 0 else t - 1]
            srcB = g1B.at[3 if t == 0 else t - 1]
            rdmaA, rdmaB = send_pair(srcA, g1A.at[t], g1A_s.at[t], rightP,
                                     srcB, g1B.at[t], g1B_s.at[t], leftP)
            rdmaA.wait()
            rdmaB.wait()
            rcA = lax.rem(pin - t + 8, 4) * PC
            rcB = lax.rem(pin + t, 4) * PC
            out_ref[pl.ds(rcA, PC), :HALF] = g1A[t].astype(jnp.float32)
            out_ref[pl.ds(rcB, PC), HALF:] = g1B[t].astype(jnp.float32)

    return pl.pallas_call(
        body,
        out_shape=jax.ShapeDtypeStruct((SQ, D_MODEL), jnp.float32),
        in_specs=[
            pl.BlockSpec(memory_space=pltpu.SMEM),
            pl.BlockSpec(memory_space=pltpu.VMEM),
            pl.BlockSpec(memory_space=pltpu.VMEM),
            pl.BlockSpec(memory_space=pl.ANY),
            pl.BlockSpec(memory_space=pl.ANY),
            pl.BlockSpec(memory_space=pltpu.VMEM),
        ],
        out_specs=pl.BlockSpec(memory_space=pltpu.VMEM),
        scratch_shapes=[
            pltpu.VMEM((SKV, H_PER, DH), jnp.float32),
            pltpu.VMEM((SKV, H_PER, DH), jnp.float32),
            pltpu.VMEM((SKV, H_PER * DH), jnp.bfloat16),
            pltpu.VMEM((SKV, H_PER * DH), jnp.bfloat16),
            pltpu.VMEM((PC, HALF), jnp.bfloat16),
            pltpu.VMEM((PC, HALF), jnp.bfloat16),
            pltpu.VMEM((ZC, HALF), jnp.bfloat16),
            pltpu.VMEM((ZC, HALF), jnp.bfloat16),
            pltpu.VMEM((3, PC, HALF), jnp.bfloat16),
            pltpu.VMEM((3, PC, HALF), jnp.bfloat16),
            pltpu.VMEM((3, ZC, HALF), jnp.bfloat16),
            pltpu.VMEM((3, ZC, HALF), jnp.bfloat16),
            pltpu.VMEM((4, ZC, HALF), jnp.bfloat16),
            pltpu.VMEM((4, ZC, HALF), jnp.bfloat16),
            pltpu.VMEM((4, PC, HALF), jnp.bfloat16),
            pltpu.VMEM((4, PC, HALF), jnp.bfloat16),
            pltpu.SemaphoreType.DMA((2,)),
            pltpu.SemaphoreType.DMA,
            pltpu.SemaphoreType.DMA,
            pltpu.SemaphoreType.DMA((3,)),
            pltpu.SemaphoreType.DMA((3,)),
            pltpu.SemaphoreType.DMA((3,)),
            pltpu.SemaphoreType.DMA((3,)),
            pltpu.SemaphoreType.DMA((3,)),
            pltpu.SemaphoreType.DMA((3,)),
            pltpu.SemaphoreType.DMA((3,)),
            pltpu.SemaphoreType.DMA((3,)),
        ],
        compiler_params=pltpu.CompilerParams(
            collective_id=0, vmem_limit_bytes=96 * 1024 * 1024),
    )(me, xb, Wq, k_hbm, v_hbm, Wo)


def kernel(x, Wq, K_ext, V_ext, Wo):
    me = lax.axis_index("i")
    xb = x[0].astype(jnp.bfloat16)
    out = _fused(xb, Wq.astype(jnp.bfloat16), K_ext[0], V_ext[0],
                 Wo.astype(jnp.bfloat16), me.astype(jnp.int32)[None])
    return out[None]


# device time: 134996 ns/iter; 3.4908x vs baseline; 1.0409x over previous
import jax
import jax.numpy as jnp
from jax import lax
from jax.experimental import pallas as pl
from jax.experimental.pallas import tpu as pltpu

N_DEV = 16
SQ = 2048
SKV = 2048
D_MODEL = 1024
H_PER = 8
DH = 128
SCALE = 0.08838834764831843
QB = 128
N_BLK = SQ // QB
HALF = D_MODEL // 2
PC = SQ // 4
ZC = PC // 4
NEG = -1e9


def _fused(xb, Wq, k_hbm, v_hbm, Wo, me):
    def body(me_ref, x_ref, wq_ref, kh_ref, vh_ref, wo_ref, out_ref,
             kf, vf, kb, vb,
             snd1A, snd1B, snd2A, snd2B, p1A, p1B, p2A, p2B,
             g2A, g2B, g1A, g1B,
             kv_sems, send_semA, send_semB, sndz_sA, sndz_sB,
             p1A_s, p1B_s, p2A_s, p2B_s, g2A_s, g2B_s, g1A_s, g1B_s):
        me = me_ref[0]
        pin = lax.rem(me, 4)
        zi = me // 4
        base = me - pin

        def plane_dev(p):
            return base + lax.rem(p + 8, 4)

        def z_dev(z):
            return lax.rem(z + 8, 4) * 4 + pin

        nbrs = [plane_dev(pin + 1), plane_dev(pin - 1),
                z_dev(zi + 1), z_dev(zi - 1)]
        barrier_sem = pltpu.get_barrier_semaphore()
        for nbr in nbrs:
            pl.semaphore_signal(
                barrier_sem, inc=1,
                device_id=(nbr,), device_id_type=pl.DeviceIdType.MESH,
            )

        kcp = pltpu.make_async_copy(
            kh_ref.at[:, pl.ds(me * H_PER, H_PER), :], kf, kv_sems.at[0])
        vcp = pltpu.make_async_copy(
            vh_ref.at[:, pl.ds(me * H_PER, H_PER), :], vf, kv_sems.at[1])
        kcp.start()
        vcp.start()
        kcp.wait()
        vcp.wait()
        kb[...] = kf[...].reshape(SKV, H_PER * DH).astype(jnp.bfloat16)
        vb[...] = vf[...].reshape(SKV, H_PER * DH).astype(jnp.bfloat16)

        pl.semaphore_wait(barrier_sem, 4)

        def qproj(b):
            x_blk = x_ref[pl.ds(b * QB, QB), :]
            qf = jnp.dot(x_blk, wq_ref[...],
                         preferred_element_type=jnp.float32)
            return (qf * SCALE).astype(jnp.bfloat16)

        def out_proj(b, ctx_f32):
            out_ref[pl.ds(b * QB, QB), :] = jnp.dot(
                ctx_f32.astype(jnp.bfloat16), wo_ref[...],
                preferred_element_type=jnp.float32)

        def dense_block0():
            q = qproj(0)
            qi = lax.broadcasted_iota(jnp.int32, (QB, 1), 0)
            ki = lax.broadcasted_iota(jnp.int32, (QB, SKV), 1)
            keep = (jnp.abs(qi - ki) <= 128) | (ki < 32) | (qi < 32)
            bias = jnp.where(keep, 0.0, NEG).astype(jnp.float32)
            ctxs = []
            for h in range(H_PER):
                q_h = q[:, h * DH:(h + 1) * DH]
                s = lax.dot_general(
                    q_h, kb[:, h * DH:(h + 1) * DH],
                    dimension_numbers=(((1,), (1,)), ((), ())),
                    preferred_element_type=jnp.float32,
                ) + bias
                e = jnp.exp(s)
                r = 1.0 / jnp.sum(e, axis=-1, keepdims=True)
                ctx = jnp.dot(e.astype(jnp.bfloat16),
                              vb[:, h * DH:(h + 1) * DH],
                              preferred_element_type=jnp.float32)
                ctxs.append(ctx * r)
            out_proj(0, jnp.concatenate(ctxs, axis=1))

        def sparse_block(b):
            q = qproj(b)
            row = lax.broadcasted_iota(jnp.int32, (QB, 1), 0)
            qi = b * QB + row
            kbs = [b - b, jnp.maximum(b - 1, 0), b,
                   jnp.minimum(b + 1, N_BLK - 1)]
            valid = [b >= 0, b != 1, b >= 0, b != N_BLK - 1]
            col = lax.broadcasted_iota(jnp.int32, (QB, QB), 1)
            biases = []
            for kb_i, ok in zip(kbs, valid):
                ki = kb_i * QB + col
                keep = ((jnp.abs(qi - ki) <= 128) | (ki < 32)
                        | (qi < 32)) & ok
                biases.append(jnp.where(keep, 0.0, NEG))
            bias = jnp.concatenate(biases, axis=1).astype(jnp.float32)

            ctxs = []
            for h in range(H_PER):
                q_h = q[:, h * DH:(h + 1) * DH]
                s_slots = []
                v_slots = []
                for kb_i in kbs:
                    s_slots.append(lax.dot_general(
                        q_h, kb[pl.ds(kb_i * QB, QB), h * DH:(h + 1) * DH],
                        dimension_numbers=(((1,), (1,)), ((), ())),
                        preferred_element_type=jnp.float32,
                    ))
                    v_slots.append(vb[pl.ds(kb_i * QB, QB),
                                      h * DH:(h + 1) * DH])
                s = jnp.concatenate(s_slots, axis=1) + bias
                e = jnp.exp(s)
                r = 1.0 / jnp.sum(e, axis=-1, keepdims=True)
                v_sel = jnp.concatenate(v_slots, axis=0)
                ctx = jnp.dot(e.astype(jnp.bfloat16), v_sel,
                              preferred_element_type=jnp.float32)
                ctxs.append(ctx * r)
            out_proj(b, jnp.concatenate(ctxs, axis=1))

        def compute_quarter(qtr):
            for i in range(4):
                b = qtr * 4 + i

                @pl.when(b > 0)
                def _():
                    sparse_block(b)

        def send_pair(srcA, dstA, semA, devA, srcB, dstB, semB, devB,
                      stageA=None, stageB=None):
            if stageA is not None:
                stageA()
            rdmaA = pltpu.make_async_remote_copy(
                src_ref=srcA, dst_ref=dstA, send_sem=send_semA,
                recv_sem=semA, device_id=(devA,),
                device_id_type=pl.DeviceIdType.MESH,
            )
            rdmaA.start()
            if stageB is not None:
                stageB()
            rdmaB = pltpu.make_async_remote_copy(
                src_ref=srcB, dst_ref=dstB, send_sem=send_semB,
                recv_sem=semB, device_id=(devB,),
                device_id_type=pl.DeviceIdType.MESH,
            )
            rdmaB.start()
            return rdmaA, rdmaB

        rightP = plane_dev(pin + 1)
        leftP = plane_dev(pin - 1)
        upZ = z_dev(zi + 1)
        downZ = z_dev(zi - 1)

        dense_block0()
        compute_quarter(pin)
        for s in range(3):
            cA = lax.rem(pin - s + 8, 4)
            cB = lax.rem(pin + s, 4)

            def _stA(cA=cA):
                snd1A[...] = out_ref[pl.ds(cA * PC, PC),
                                     :HALF].astype(jnp.bfloat16)

            def _stB(cB=cB):
                snd1B[...] = out_ref[pl.ds(cB * PC, PC),
                                     HALF:].astype(jnp.bfloat16)

            rdmaA, rdmaB = send_pair(
                snd1A, p1A.at[s], p1A_s.at[s], rightP,
                snd1B, p1B.at[s], p1B_s.at[s], leftP,
                stageA=_stA, stageB=_stB)

            if s == 0:
                compute_quarter(lax.rem(pin + 3, 4))
                compute_quarter(lax.rem(pin + 1, 4))
            elif s == 1:
                compute_quarter(lax.rem(pin + 2, 4))

            rdmaA.wait()
            rdmaB.wait()
            rA = lax.rem(pin - s - 1 + 8, 4)
            rB = lax.rem(pin + s + 1, 4)
            out_ref[pl.ds(rA * PC, PC), :HALF] = (
                out_ref[pl.ds(rA * PC, PC), :HALF] + p1A[s].astype(jnp.float32))
            out_ref[pl.ds(rB * PC, PC), HALF:] = (
                out_ref[pl.ds(rB * PC, PC), HALF:] + p1B[s].astype(jnp.float32))

        qA = lax.rem(pin + 1, 4) * PC
        qB = lax.rem(pin + 3, 4) * PC

        rdmas = []
        for r in range(1, 4):
            zt = lax.rem(zi + r, 4)
            slot = 3 - r
            snd2A[r - 1] = out_ref[pl.ds(qA + zt * ZC, ZC),
                                   :HALF].astype(jnp.bfloat16)
            snd2B[r - 1] = out_ref[pl.ds(qB + zt * ZC, ZC),
                                   HALF:].astype(jnp.bfloat16)
            rdmaA = pltpu.make_async_remote_copy(
                src_ref=snd2A.at[r - 1], dst_ref=p2A.at[slot],
                send_sem=sndz_sA.at[r - 1], recv_sem=p2A_s.at[slot],
                device_id=(z_dev(zi + r),),
                device_id_type=pl.DeviceIdType.MESH,
            )
            rdmaB = pltpu.make_async_remote_copy(
                src_ref=snd2B.at[r - 1], dst_ref=p2B.at[slot],
                send_sem=sndz_sB.at[r - 1], recv_sem=p2B_s.at[slot],
                device_id=(z_dev(zi + r),),
                device_id_type=pl.DeviceIdType.MESH,
            )
            rdmaA.start()
            rdmaB.start()
            rdmas += [rdmaA, rdmaB]
        for rd in rdmas:
            rd.wait()
        accA = (p2A[0].astype(jnp.float32) + p2A[1].astype(jnp.float32)
                + p2A[2].astype(jnp.float32))
        accB = (p2B[0].astype(jnp.float32) + p2B[1].astype(jnp.float32)
                + p2B[2].astype(jnp.float32))
        out_ref[pl.ds(qA + zi * ZC, ZC), :HALF] = (
            out_ref[pl.ds(qA + zi * ZC, ZC), :HALF] + accA)
        out_ref[pl.ds(qB + zi * ZC, ZC), HALF:] = (
            out_ref[pl.ds(qB + zi * ZC, ZC), HALF:] + accB)

        g2A[3] = out_ref[pl.ds(qA + zi * ZC, ZC), :HALF].astype(jnp.bfloat16)
        g2B[3] = out_ref[pl.ds(qB + zi * ZC, ZC), HALF:].astype(jnp.bfloat16)
        rdmas = []
        for r in range(1, 4):
            slot = 3 - r
            rdmaA = pltpu.make_async_remote_copy(
                src_ref=g2A.at[3], dst_ref=g2A.at[slot],
                send_sem=sndz_sA.at[r - 1], recv_sem=g2A_s.at[slot],
                device_id=(z_dev(zi + r),),
                device_id_type=pl.DeviceIdType.MESH,
            )
            rdmaB = pltpu.make_async_remote_copy(
                src_ref=g2B.at[3], dst_ref=g2B.at[slot],
                send_sem=sndz_sB.at[r - 1], recv_sem=g2B_s.at[slot],
                device_id=(z_dev(zi + r),),
                device_id_type=pl.DeviceIdType.MESH,
            )
            rdmaA.start()
            rdmaB.start()
            rdmas += [rdmaA, rdmaB]
        for rd in rdmas:
            rd.wait()
        for r in range(1, 4):
            zc = lax.rem(zi + r, 4)
            out_ref[pl.ds(qA + zc * ZC, ZC), :HALF] = (
                g2A[r - 1].astype(jnp.float32))
            out_ref[pl.ds(qB + zc * ZC, ZC), HALF:] = (
                g2B[r - 1].astype(jnp.float32))

        g1A[3] = out_ref[pl.ds(qA, PC), :HALF].astype(jnp.bfloat16)
        g1B[3] = out_ref[pl.ds(qB, PC), HALF:].astype(jnp.bfloat16)
        for t in range(3):
            srcA = g1A.at[3 if t == 0 else t - 1]
            srcB = g1B.at[3 if t == 0 else t - 1]
            rdmaA, rdmaB = send_pair(srcA, g1A.at[t], g1A_s.at[t], rightP,
                                     srcB, g1B.at[t], g1B_s.at[t], leftP)
            rdmaA.wait()
            rdmaB.wait()
            rcA = lax.rem(pin - t + 8, 4) * PC
            rcB = lax.rem(pin + t, 4) * PC
            out_ref[pl.ds(rcA, PC), :HALF] = g1A[t].astype(jnp.float32)
            out_ref[pl.ds(rcB, PC), HALF:] = g1B[t].astype(jnp.float32)

    return pl.pallas_call(
        body,
        out_shape=jax.ShapeDtypeStruct((SQ, D_MODEL), jnp.float32),
        in_specs=[
            pl.BlockSpec(memory_space=pltpu.SMEM),
            pl.BlockSpec(memory_space=pltpu.VMEM),
            pl.BlockSpec(memory_space=pltpu.VMEM),
            pl.BlockSpec(memory_space=pl.ANY),
            pl.BlockSpec(memory_space=pl.ANY),
            pl.BlockSpec(memory_space=pltpu.VMEM),
        ],
        out_specs=pl.BlockSpec(memory_space=pltpu.VMEM),
        scratch_shapes=[
            pltpu.VMEM((SKV, H_PER, DH), jnp.float32),
            pltpu.VMEM((SKV, H_PER, DH), jnp.float32),
            pltpu.VMEM((SKV, H_PER * DH), jnp.bfloat16),
            pltpu.VMEM((SKV, H_PER * DH), jnp.bfloat16),
            pltpu.VMEM((PC, HALF), jnp.bfloat16),
            pltpu.VMEM((PC, HALF), jnp.bfloat16),
            pltpu.VMEM((3, ZC, HALF), jnp.bfloat16),
            pltpu.VMEM((3, ZC, HALF), jnp.bfloat16),
            pltpu.VMEM((3, PC, HALF), jnp.bfloat16),
            pltpu.VMEM((3, PC, HALF), jnp.bfloat16),
            pltpu.VMEM((3, ZC, HALF), jnp.bfloat16),
            pltpu.VMEM((3, ZC, HALF), jnp.bfloat16),
            pltpu.VMEM((4, ZC, HALF), jnp.bfloat16),
            pltpu.VMEM((4, ZC, HALF), jnp.bfloat16),
            pltpu.VMEM((4, PC, HALF), jnp.bfloat16),
            pltpu.VMEM((4, PC, HALF), jnp.bfloat16),
            pltpu.SemaphoreType.DMA((2,)),
            pltpu.SemaphoreType.DMA,
            pltpu.SemaphoreType.DMA,
            pltpu.SemaphoreType.DMA((3,)),
            pltpu.SemaphoreType.DMA((3,)),
            pltpu.SemaphoreType.DMA((3,)),
            pltpu.SemaphoreType.DMA((3,)),
            pltpu.SemaphoreType.DMA((3,)),
            pltpu.SemaphoreType.DMA((3,)),
            pltpu.SemaphoreType.DMA((3,)),
            pltpu.SemaphoreType.DMA((3,)),
            pltpu.SemaphoreType.DMA((3,)),
            pltpu.SemaphoreType.DMA((3,)),
        ],
        compiler_params=pltpu.CompilerParams(
            collective_id=0, vmem_limit_bytes=96 * 1024 * 1024),
    )(me, xb, Wq, k_hbm, v_hbm, Wo)


def kernel(x, Wq, K_ext, V_ext, Wo):
    me = lax.axis_index("i")
    xb = x[0].astype(jnp.bfloat16)
    out = _fused(xb, Wq.astype(jnp.bfloat16), K_ext[0], V_ext[0],
                 Wo.astype(jnp.bfloat16), me.astype(jnp.int32)[None])
    return out[None]


# device time: 131426 ns/iter; 3.5856x vs baseline; 1.0272x over previous
import jax
import jax.numpy as jnp
from jax import lax
from jax.experimental import pallas as pl
from jax.experimental.pallas import tpu as pltpu

N_DEV = 16
SQ = 2048
SKV = 2048
D_MODEL = 1024
H_PER = 8
DH = 128
SCALE = 0.08838834764831843
QB = 128
N_BLK = SQ // QB
HALF = D_MODEL // 2
PC = SQ // 4
ZC = PC // 4
NEG = -1e9


def _fused(xb, Wq, k_hbm, v_hbm, Wo, me):
    def body(me_ref, x_ref, wq_ref, kh_ref, vh_ref, wo_ref, out_ref,
             kf, vf, kb, vb,
             snd1A, snd1B, snd2A, snd2B, p1A, p1B, p2A, p2B,
             g2A, g2B, g1A, g1B,
             kv_sems, send_semA, send_semB, sndz_sA, sndz_sB,
             p1A_s, p1B_s, p2A_s, p2B_s, g2A_s, g2B_s, g1A_s, g1B_s):
        me = me_ref[0]
        pin = lax.rem(me, 4)
        zi = me // 4
        base = me - pin

        def plane_dev(p):
            return base + lax.rem(p + 8, 4)

        def z_dev(z):
            return lax.rem(z + 8, 4) * 4 + pin

        nbrs = [plane_dev(pin + 1), plane_dev(pin - 1),
                z_dev(zi + 1), z_dev(zi - 1)]
        barrier_sem = pltpu.get_barrier_semaphore()
        for nbr in nbrs:
            pl.semaphore_signal(
                barrier_sem, inc=1,
                device_id=(nbr,), device_id_type=pl.DeviceIdType.MESH,
            )

        kcp = pltpu.make_async_copy(
            kh_ref.at[:, pl.ds(me * H_PER, H_PER), :], kf, kv_sems.at[0])
        vcp = pltpu.make_async_copy(
            vh_ref.at[:, pl.ds(me * H_PER, H_PER), :], vf, kv_sems.at[1])
        kcp.start()
        vcp.start()
        kcp.wait()
        vcp.wait()
        kb[...] = kf[...].reshape(SKV, H_PER * DH).astype(jnp.bfloat16)
        vb[...] = vf[...].reshape(SKV, H_PER * DH).astype(jnp.bfloat16)

        pl.semaphore_wait(barrier_sem, 4)

        def qproj(b):
            x_blk = x_ref[pl.ds(b * QB, QB), :]
            qf = jnp.dot(x_blk, wq_ref[...],
                         preferred_element_type=jnp.float32)
            return (qf * SCALE).astype(jnp.bfloat16)

        def out_proj(b, ctx_f32):
            out_ref[pl.ds(b * QB, QB), :] = jnp.dot(
                ctx_f32.astype(jnp.bfloat16), wo_ref[...],
                preferred_element_type=jnp.float32)

        def dense_block0():
            q = qproj(0)
            qi = lax.broadcasted_iota(jnp.int32, (QB, 1), 0)
            ki = lax.broadcasted_iota(jnp.int32, (QB, SKV), 1)
            keep = (jnp.abs(qi - ki) <= 128) | (ki < 32) | (qi < 32)
            bias = jnp.where(keep, 0.0, NEG).astype(jnp.float32)
            ctxs = []
            for h in range(H_PER):
                q_h = q[:, h * DH:(h + 1) * DH]
                s = lax.dot_general(
                    q_h, kb[:, h * DH:(h + 1) * DH],
                    dimension_numbers=(((1,), (1,)), ((), ())),
                    preferred_element_type=jnp.float32,
                ) + bias
                e = jnp.exp(s)
                r = 1.0 / jnp.sum(e, axis=-1, keepdims=True)
                ctx = jnp.dot(e.astype(jnp.bfloat16),
                              vb[:, h * DH:(h + 1) * DH],
                              preferred_element_type=jnp.float32)
                ctxs.append(ctx * r)
            out_proj(0, jnp.concatenate(ctxs, axis=1))

        def sparse_block(b):
            q = qproj(b)
            row = lax.broadcasted_iota(jnp.int32, (QB, 1), 0)
            qi = b * QB + row
            kbs = [b - b, jnp.maximum(b - 1, 0), b,
                   jnp.minimum(b + 1, N_BLK - 1)]
            valid = [b >= 0, b != 1, b >= 0, b != N_BLK - 1]
            col = lax.broadcasted_iota(jnp.int32, (QB, QB), 1)
            biases = []
            for kb_i, ok in zip(kbs, valid):
                ki = kb_i * QB + col
                keep = ((jnp.abs(qi - ki) <= 128) | (ki < 32)
                        | (qi < 32)) & ok
                biases.append(jnp.where(keep, 0.0, NEG))
            bias = jnp.concatenate(biases, axis=1).astype(jnp.float32)

            ctxs = []
            for h in range(H_PER):
                q_h = q[:, h * DH:(h + 1) * DH]
                s_slots = []
                v_slots = []
                for kb_i in kbs:
                    s_slots.append(lax.dot_general(
                        q_h, kb[pl.ds(kb_i * QB, QB), h * DH:(h + 1) * DH],
                        dimension_numbers=(((1,), (1,)), ((), ())),
                        preferred_element_type=jnp.float32,
                    ))
                    v_slots.append(vb[pl.ds(kb_i * QB, QB),
                                      h * DH:(h + 1) * DH])
                s = jnp.concatenate(s_slots, axis=1) + bias
                e = jnp.exp(s)
                r = 1.0 / jnp.sum(e, axis=-1, keepdims=True)
                v_sel = jnp.concatenate(v_slots, axis=0)
                ctx = jnp.dot(e.astype(jnp.bfloat16), v_sel,
                              preferred_element_type=jnp.float32)
                ctxs.append(ctx * r)
            out_proj(b, jnp.concatenate(ctxs, axis=1))

        def compute_quarter(qtr):
            for i in range(4):
                b = qtr * 4 + i

                @pl.when(b > 0)
                def _():
                    sparse_block(b)

        def send_pair(srcA, dstA, semA, devA, srcB, dstB, semB, devB,
                      stageA=None, stageB=None):
            if stageA is not None:
                stageA()
            rdmaA = pltpu.make_async_remote_copy(
                src_ref=srcA, dst_ref=dstA, send_sem=send_semA,
                recv_sem=semA, device_id=(devA,),
                device_id_type=pl.DeviceIdType.MESH,
            )
            rdmaA.start()
            if stageB is not None:
                stageB()
            rdmaB = pltpu.make_async_remote_copy(
                src_ref=srcB, dst_ref=dstB, send_sem=send_semB,
                recv_sem=semB, device_id=(devB,),
                device_id_type=pl.DeviceIdType.MESH,
            )
            rdmaB.start()
            return rdmaA, rdmaB

        rightP = plane_dev(pin + 1)
        leftP = plane_dev(pin - 1)
        upZ = z_dev(zi + 1)
        downZ = z_dev(zi - 1)

        dense_block0()
        compute_quarter(pin)
        for s in range(3):
            cA = lax.rem(pin - s + 8, 4)
            cB = lax.rem(pin + s, 4)

            def _stA(cA=cA):
                snd1A[...] = out_ref[pl.ds(cA * PC, PC),
                                     :HALF].astype(jnp.bfloat16)

            def _stB(cB=cB):
                snd1B[...] = out_ref[pl.ds(cB * PC, PC),
                                     HALF:].astype(jnp.bfloat16)

            rdmaA, rdmaB = send_pair(
                snd1A, p1A.at[s], p1A_s.at[s], rightP,
                snd1B, p1B.at[s], p1B_s.at[s], leftP,
                stageA=_stA, stageB=_stB)

            if s == 0:
                compute_quarter(lax.rem(pin + 3, 4))
                compute_quarter(lax.rem(pin + 1, 4))
            elif s == 1:
                compute_quarter(lax.rem(pin + 2, 4))

            rdmaA.wait()
            rdmaB.wait()
            rA = lax.rem(pin - s - 1 + 8, 4)
            rB = lax.rem(pin + s + 1, 4)
            out_ref[pl.ds(rA * PC, PC), :HALF] = (
                out_ref[pl.ds(rA * PC, PC), :HALF] + p1A[s].astype(jnp.float32))
            out_ref[pl.ds(rB * PC, PC), HALF:] = (
                out_ref[pl.ds(rB * PC, PC), HALF:] + p1B[s].astype(jnp.float32))

        qA = lax.rem(pin + 1, 4) * PC
        qB = lax.rem(pin + 3, 4) * PC

        rdmas = []
        for r in range(1, 4):
            zt = lax.rem(zi + r, 4)
            slot = 3 - r
            snd2A[r - 1] = out_ref[pl.ds(qA + zt * ZC, ZC),
                                   :HALF].astype(jnp.bfloat16)
            snd2B[r - 1] = out_ref[pl.ds(qB + zt * ZC, ZC),
                                   HALF:].astype(jnp.bfloat16)
            rdmaA = pltpu.make_async_remote_copy(
                src_ref=snd2A.at[r - 1], dst_ref=p2A.at[slot],
                send_sem=sndz_sA.at[r - 1], recv_sem=p2A_s.at[slot],
                device_id=(z_dev(zi + r),),
                device_id_type=pl.DeviceIdType.MESH,
            )
            rdmaB = pltpu.make_async_remote_copy(
                src_ref=snd2B.at[r - 1], dst_ref=p2B.at[slot],
                send_sem=sndz_sB.at[r - 1], recv_sem=p2B_s.at[slot],
                device_id=(z_dev(zi + r),),
                device_id_type=pl.DeviceIdType.MESH,
            )
            rdmaA.start()
            rdmaB.start()
            rdmas += [rdmaA, rdmaB]
        for rd in rdmas:
            rd.wait()
        accA = (p2A[0].astype(jnp.float32) + p2A[1].astype(jnp.float32)
                + p2A[2].astype(jnp.float32))
        accB = (p2B[0].astype(jnp.float32) + p2B[1].astype(jnp.float32)
                + p2B[2].astype(jnp.float32))
        out_ref[pl.ds(qA + zi * ZC, ZC), :HALF] = (
            out_ref[pl.ds(qA + zi * ZC, ZC), :HALF] + accA)
        out_ref[pl.ds(qB + zi * ZC, ZC), HALF:] = (
            out_ref[pl.ds(qB + zi * ZC, ZC), HALF:] + accB)

        g2A[3] = out_ref[pl.ds(qA + zi * ZC, ZC), :HALF].astype(jnp.bfloat16)
        g2B[3] = out_ref[pl.ds(qB + zi * ZC, ZC), HALF:].astype(jnp.bfloat16)
        rdmas = []
        for r in range(1, 4):
            slot = 3 - r
            rdmaA = pltpu.make_async_remote_copy(
                src_ref=g2A.at[3], dst_ref=g2A.at[slot],
                send_sem=sndz_sA.at[r - 1], recv_sem=g2A_s.at[slot],
                device_id=(z_dev(zi + r),),
                device_id_type=pl.DeviceIdType.MESH,
            )
            rdmaB = pltpu.make_async_remote_copy(
                src_ref=g2B.at[3], dst_ref=g2B.at[slot],
                send_sem=sndz_sB.at[r - 1], recv_sem=g2B_s.at[slot],
                device_id=(z_dev(zi + r),),
                device_id_type=pl.DeviceIdType.MESH,
            )
            rdmaA.start()
            rdmaB.start()
            rdmas += [rdmaA, rdmaB]
        for rd in rdmas:
            rd.wait()
        for r in range(1, 4):
            zc = lax.rem(zi + r, 4)
            out_ref[pl.ds(qA + zc * ZC, ZC), :HALF] = (
                g2A[r - 1].astype(jnp.float32))
            out_ref[pl.ds(qB + zc * ZC, ZC), HALF:] = (
                g2B[r - 1].astype(jnp.float32))

        HPC = PC // 2
        g1A[3] = out_ref[pl.ds(qA, PC), :HALF].astype(jnp.bfloat16)
        g1B[3] = out_ref[pl.ds(qB, PC), HALF:].astype(jnp.bfloat16)
        pend = {}

        def _p4_send(t, sub):
            src_slot = 3 if t == 0 else t - 1
            rdmaA = pltpu.make_async_remote_copy(
                src_ref=g1A.at[src_slot, pl.ds(sub * HPC, HPC), :],
                dst_ref=g1A.at[t, pl.ds(sub * HPC, HPC), :],
                send_sem=sndz_sA.at[sub], recv_sem=g1A_s.at[t * 2 + sub],
                device_id=(rightP,), device_id_type=pl.DeviceIdType.MESH,
            )
            rdmaB = pltpu.make_async_remote_copy(
                src_ref=g1B.at[src_slot, pl.ds(sub * HPC, HPC), :],
                dst_ref=g1B.at[t, pl.ds(sub * HPC, HPC), :],
                send_sem=sndz_sB.at[sub], recv_sem=g1B_s.at[t * 2 + sub],
                device_id=(leftP,), device_id_type=pl.DeviceIdType.MESH,
            )
            rdmaA.start()
            rdmaB.start()
            pend[(t, sub)] = (rdmaA, rdmaB)

        for t in range(3):
            for sub in (0, 1):
                if t > 0:
                    for rd in pend[(t - 1, sub)]:
                        rd.wait()
                _p4_send(t, sub)
        for sub in (0, 1):
            for rd in pend[(2, sub)]:
                rd.wait()
        for t in range(3):
            rcA = lax.rem(pin - t + 8, 4) * PC
            rcB = lax.rem(pin + t, 4) * PC
            out_ref[pl.ds(rcA, PC), :HALF] = g1A[t].astype(jnp.float32)
            out_ref[pl.ds(rcB, PC), HALF:] = g1B[t].astype(jnp.float32)

    return pl.pallas_call(
        body,
        out_shape=jax.ShapeDtypeStruct((SQ, D_MODEL), jnp.float32),
        in_specs=[
            pl.BlockSpec(memory_space=pltpu.SMEM),
            pl.BlockSpec(memory_space=pltpu.VMEM),
            pl.BlockSpec(memory_space=pltpu.VMEM),
            pl.BlockSpec(memory_space=pl.ANY),
            pl.BlockSpec(memory_space=pl.ANY),
            pl.BlockSpec(memory_space=pltpu.VMEM),
        ],
        out_specs=pl.BlockSpec(memory_space=pltpu.VMEM),
        scratch_shapes=[
            pltpu.VMEM((SKV, H_PER, DH), jnp.float32),
            pltpu.VMEM((SKV, H_PER, DH), jnp.float32),
            pltpu.VMEM((SKV, H_PER * DH), jnp.bfloat16),
            pltpu.VMEM((SKV, H_PER * DH), jnp.bfloat16),
            pltpu.VMEM((PC, HALF), jnp.bfloat16),
            pltpu.VMEM((PC, HALF), jnp.bfloat16),
            pltpu.VMEM((3, ZC, HALF), jnp.bfloat16),
            pltpu.VMEM((3, ZC, HALF), jnp.bfloat16),
            pltpu.VMEM((3, PC, HALF), jnp.bfloat16),
            pltpu.VMEM((3, PC, HALF), jnp.bfloat16),
            pltpu.VMEM((3, ZC, HALF), jnp.bfloat16),
            pltpu.VMEM((3, ZC, HALF), jnp.bfloat16),
            pltpu.VMEM((4, ZC, HALF), jnp.bfloat16),
            pltpu.VMEM((4, ZC, HALF), jnp.bfloat16),
            pltpu.VMEM((4, PC, HALF), jnp.bfloat16),
            pltpu.VMEM((4, PC, HALF), jnp.bfloat16),
            pltpu.SemaphoreType.DMA((2,)),
            pltpu.SemaphoreType.DMA,
            pltpu.SemaphoreType.DMA,
            pltpu.SemaphoreType.DMA((3,)),
            pltpu.SemaphoreType.DMA((3,)),
            pltpu.SemaphoreType.DMA((3,)),
            pltpu.SemaphoreType.DMA((3,)),
            pltpu.SemaphoreType.DMA((3,)),
            pltpu.SemaphoreType.DMA((3,)),
            pltpu.SemaphoreType.DMA((3,)),
            pltpu.SemaphoreType.DMA((3,)),
            pltpu.SemaphoreType.DMA((6,)),
            pltpu.SemaphoreType.DMA((6,)),
        ],
        compiler_params=pltpu.CompilerParams(
            collective_id=0, vmem_limit_bytes=96 * 1024 * 1024),
    )(me, xb, Wq, k_hbm, v_hbm, Wo)


def kernel(x, Wq, K_ext, V_ext, Wo):
    me = lax.axis_index("i")
    xb = x[0].astype(jnp.bfloat16)
    out = _fused(xb, Wq.astype(jnp.bfloat16), K_ext[0], V_ext[0],
                 Wo.astype(jnp.bfloat16), me.astype(jnp.int32)[None])
    return out[None]
